# Initial kernel scaffold; baseline (speedup 1.0000x reference)
import jax
import jax.numpy as jnp
from jax.experimental import pallas as pl


def kernel(x, router_w, gate_up_proj, down_proj):
    raise NotImplementedError("write your pallas kernel here")


# speedup vs baseline: 2.1860x; 2.1860x over previous
import functools

import jax
import jax.numpy as jnp
from jax.experimental import pallas as pl
from jax.experimental.pallas import tpu as pltpu


def _router_kernel(x_ref, w_ref, out_ref, *, n_exp):
    logits = jnp.dot(x_ref[:], w_ref[:], preferred_element_type=jnp.float32)
    m = jnp.max(logits, axis=-1, keepdims=True)
    ex = jnp.exp(logits - m)
    probs = ex / jnp.sum(ex, axis=-1, keepdims=True)
    ids = jax.lax.broadcasted_iota(jnp.int32, probs.shape, 1)
    big = jnp.int32(n_exp)
    v1 = jnp.max(probs, axis=-1, keepdims=True)
    i1 = jnp.min(jnp.where(probs == v1, ids, big), axis=-1, keepdims=True)
    m1 = ids == i1
    p2 = jnp.where(m1, -jnp.inf, probs)
    v2 = jnp.max(p2, axis=-1, keepdims=True)
    i2 = jnp.min(jnp.where(p2 == v2, ids, big), axis=-1, keepdims=True)
    m2 = ids == i2
    s = v1 + v2
    out_ref[:] = jnp.where(m1, v1 / s, jnp.where(m2, v2 / s, 0.0))


def _expert_kernel(x_ref, gu_ref, dp_ref, rw_ref, out_ref, *, ff):
    e = pl.program_id(1)
    h = jnp.dot(x_ref[:], gu_ref[0], preferred_element_type=jnp.float32)
    g = h[:, :ff]
    u = h[:, ff:]
    act = (g * jax.nn.sigmoid(g) * u).astype(jnp.bfloat16)
    y = jnp.dot(act, dp_ref[0], preferred_element_type=jnp.float32)
    rw = rw_ref[:]
    ids = jax.lax.broadcasted_iota(jnp.int32, rw.shape, 1)
    w = jnp.sum(jnp.where(ids == e, rw, 0.0), axis=1, keepdims=True)
    contrib = w * y

    @pl.when(e == 0)
    def _init():
        out_ref[:] = contrib

    @pl.when(e > 0)
    def _acc():
        out_ref[:] += contrib


def kernel(x, router_w, gate_up_proj, down_proj):
    n_tok, d_model = x.shape
    n_exp = gate_up_proj.shape[0]
    ff = down_proj.shape[1]

    bm_r = min(2048, n_tok)
    rw = pl.pallas_call(
        functools.partial(_router_kernel, n_exp=n_exp),
        grid=(n_tok // bm_r,),
        in_specs=[
            pl.BlockSpec((bm_r, d_model), lambda i: (i, 0)),
            pl.BlockSpec((d_model, n_exp), lambda i: (0, 0)),
        ],
        out_specs=pl.BlockSpec((bm_r, n_exp), lambda i: (i, 0)),
        out_shape=jax.ShapeDtypeStruct((n_tok, n_exp), jnp.float32),
    )(x, router_w)

    xb = x.astype(jnp.bfloat16)
    gu = gate_up_proj.astype(jnp.bfloat16)
    dp = down_proj.astype(jnp.bfloat16)

    bm = min(1024, n_tok)
    nb = n_tok // bm
    out = pl.pallas_call(
        functools.partial(_expert_kernel, ff=ff),
        grid=(nb, n_exp),
        in_specs=[
            pl.BlockSpec((bm, d_model), lambda i, e: (i, 0)),
            pl.BlockSpec((1, d_model, 2 * ff), lambda i, e: (e, 0, 0)),
            pl.BlockSpec((1, ff, d_model), lambda i, e: (e, 0, 0)),
            pl.BlockSpec((bm, n_exp), lambda i, e: (i, 0)),
        ],
        out_specs=pl.BlockSpec((bm, d_model), lambda i, e: (i, 0)),
        out_shape=jax.ShapeDtypeStruct((n_tok, d_model), jnp.float32),
        compiler_params=pltpu.CompilerParams(
            dimension_semantics=("parallel", "arbitrary"),
        ),
    )(xb, gu, dp, rw)
    return out
